# baseline (speedup 1.0000x reference)
import functools

import jax
import jax.numpy as jnp
from jax import lax
from jax.experimental import pallas as pl
from jax.experimental.pallas import tpu as pltpu
from jax.experimental.pallas import tpu_sc as plsc

NC = 2
NS = 16
NW = NC * NS
L = 16

CHUNK = 200
NBUF = 4
G = 3


def _gather_grid(b_per_w, n_chunks, chunk, D):
    mesh = plsc.VectorSubcoreMesh(core_axis_name="c", subcore_axis_name="s")

    @functools.partial(
        pl.kernel,
        mesh=mesh,
        out_type=jax.ShapeDtypeStruct((NW * b_per_w, D), jnp.float32),
        scratch_types=(
            [pltpu.VMEM((b_per_w,), jnp.int32)]
            + [pltpu.VMEM((chunk, D), jnp.float32) for _ in range(NBUF)]
            + [pltpu.SemaphoreType.DMA for _ in range(1 + NBUF + NBUF)]
        ),
    )
    def k(atom_hbm, idx_hbm, out_hbm, idx_s, *bufs_and_sems):
        rows_v = bufs_and_sems[:NBUF]
        cs = bufs_and_sems[NBUF]
        gsem = bufs_and_sems[NBUF + 1:NBUF + 1 + NBUF]
        osem = bufs_and_sems[NBUF + 1 + NBUF:]

        wid = lax.axis_index("s") * NC + lax.axis_index("c")
        base_w = wid * b_per_w

        def out_slice(j):
            return out_hbm.at[pl.ds(base_w + j * chunk, chunk), :]

        def out_start(j):
            pltpu.async_copy(rows_v[j % NBUF], out_slice(j), osem[j % NBUF])

        def out_wait(j):
            pltpu.make_async_copy(
                rows_v[j % NBUF], out_slice(j), osem[j % NBUF]).wait()

        def gather_start(j):
            pltpu.async_copy(
                atom_hbm.at[idx_s.at[pl.ds(j * chunk, chunk)]],
                rows_v[j % NBUF], gsem[j % NBUF])

        def gather_wait(j):
            pltpu.make_async_copy(
                atom_hbm.at[idx_s.at[pl.ds(j * chunk, chunk)]],
                rows_v[j % NBUF], gsem[j % NBUF]).wait()

        idx_src = idx_hbm.at[pl.ds(base_w, b_per_w)]
        pltpu.async_copy(idx_src, idx_s, cs)
        pltpu.make_async_copy(idx_src, idx_s, cs).wait()

        for j in range(n_chunks):
            if j >= NBUF:
                out_wait(j - NBUF)
            gather_start(j)
            if j >= G:
                gather_wait(j - G)
                out_start(j - G)

        for j in range(max(0, n_chunks - G), n_chunks):
            gather_wait(j)
            out_start(j)
        for j in range(max(0, n_chunks - NBUF), n_chunks):
            out_wait(j)

    return k


def kernel(atom_matrix, connectivity):
    V, D = atom_matrix.shape
    B = connectivity.shape[0]
    assert B % NW == 0
    b_per_w = B // NW
    chunk = CHUNK
    assert b_per_w % chunk == 0 and chunk % 8 == 0
    n_chunks = b_per_w // chunk
    idx = connectivity[:, 1].astype(jnp.int32)
    return _gather_grid(b_per_w, n_chunks, chunk, D)(atom_matrix, idx)


# speedup vs baseline: 1.4452x; 1.4452x over previous
import functools

import jax
import jax.numpy as jnp
from jax import lax
from jax.experimental import pallas as pl
from jax.experimental.pallas import tpu as pltpu
from jax.experimental.pallas import tpu_sc as plsc

NC = 2
NS = 16
NW = NC * NS
L = 16


def _gather_grid(V, b_per_w, n_chunks, chunk, D):
    mesh = plsc.VectorSubcoreMesh(core_axis_name="c", subcore_axis_name="s")

    @functools.partial(
        pl.kernel,
        mesh=mesh,
        out_type=jax.ShapeDtypeStruct((NW * b_per_w, D), jnp.float32),
        scratch_types=[
            pltpu.VMEM_SHARED((V, D), jnp.float32),
            pltpu.VMEM((b_per_w,), jnp.int32),
            pltpu.VMEM((chunk, D), jnp.float32),
            pltpu.VMEM((chunk, D), jnp.float32),
            pltpu.SemaphoreType.DMA,
            pltpu.SemaphoreType.DMA,
            pltpu.SemaphoreType.DMA,
            pltpu.SemaphoreType.DMA,
            pltpu.SemaphoreType.DMA,
            pltpu.SemaphoreType.DMA,
        ],
    )
    def k(atom_hbm, idx_hbm, out_hbm,
          table_sh, idx_s, r0, r1, ts, cs, gs0, gs1, os0, os1):
        rows_v = (r0, r1)
        gsem = (gs0, gs1)
        osem = (os0, os1)

        sid = lax.axis_index("s")
        wid = sid * NC + lax.axis_index("c")
        base_w = wid * b_per_w

        def out_slice(j):
            return out_hbm.at[pl.ds(base_w + j * chunk, chunk), :]

        def out_start(j):
            pltpu.async_copy(rows_v[j % 2], out_slice(j), osem[j % 2])

        def out_wait(j):
            pltpu.make_async_copy(rows_v[j % 2], out_slice(j), osem[j % 2]).wait()

        def gather_start(j):
            pltpu.async_copy(
                table_sh.at[idx_s.at[pl.ds(j * chunk, chunk)]],
                rows_v[j % 2], gsem[j % 2])

        def gather_wait(j):
            pltpu.make_async_copy(
                table_sh.at[idx_s.at[pl.ds(j * chunk, chunk)]],
                rows_v[j % 2], gsem[j % 2]).wait()

        idx_src = idx_hbm.at[pl.ds(base_w, b_per_w)]
        pltpu.async_copy(idx_src, idx_s, cs)

        @pl.when(sid == 0)
        def _():
            pltpu.async_copy(atom_hbm, table_sh, ts)
            pltpu.make_async_copy(atom_hbm, table_sh, ts).wait()

        pltpu.make_async_copy(idx_src, idx_s, cs).wait()
        plsc.subcore_barrier()

        for j in range(n_chunks):
            if j >= 2:
                out_wait(j - 2)
            gather_start(j)
            if j >= 1:
                gather_wait(j - 1)
                out_start(j - 1)

        gather_wait(n_chunks - 1)
        out_start(n_chunks - 1)
        if n_chunks >= 2:
            out_wait(n_chunks - 2)
        out_wait(n_chunks - 1)

    return k


def kernel(atom_matrix, connectivity):
    V, D = atom_matrix.shape
    B = connectivity.shape[0]
    assert B % NW == 0
    b_per_w = B // NW
    chunk = 80
    assert b_per_w % chunk == 0 and chunk % L == 0
    n_chunks = b_per_w // chunk
    idx = connectivity[:, 1].astype(jnp.int32)
    return _gather_grid(V, b_per_w, n_chunks, chunk, D)(atom_matrix, idx)


# speedup vs baseline: 1.4938x; 1.0336x over previous
import functools

import jax
import jax.numpy as jnp
from jax import lax
from jax.experimental import pallas as pl
from jax.experimental.pallas import tpu as pltpu
from jax.experimental.pallas import tpu_sc as plsc

NC = 2
NS = 16
NW = NC * NS
L = 16

CHUNK = 80
NBUF = 4
NBI = 8
G = 3


def _gather_grid(V, b_per_w, n_chunks, chunk, D):
    mesh = plsc.VectorSubcoreMesh(core_axis_name="c", subcore_axis_name="s")

    @functools.partial(
        pl.kernel,
        mesh=mesh,
        out_type=jax.ShapeDtypeStruct((NW * b_per_w, D), jnp.float32),
        scratch_types=(
            [pltpu.VMEM_SHARED((V, D), jnp.float32)]
            + [pltpu.VMEM((chunk,), jnp.int32) for _ in range(NBI)]
            + [pltpu.VMEM((chunk, D), jnp.float32) for _ in range(NBUF)]
            + [pltpu.SemaphoreType.DMA for _ in range(1 + NBI + 2 * NBUF)]
        ),
    )
    def k(atom_hbm, idx_hbm, out_hbm, table_sh, *rest):
        idx_v = rest[:NBI]
        rows_v = rest[NBI:NBI + NBUF]
        ts = rest[NBI + NBUF]
        csem = rest[NBI + NBUF + 1:NBI + NBUF + 1 + NBI]
        gsem = rest[NBI + NBUF + 1 + NBI:NBI + NBUF + 1 + NBI + NBUF]
        osem = rest[NBI + NBUF + 1 + NBI + NBUF:]

        sid = lax.axis_index("s")
        wid = sid * NC + lax.axis_index("c")
        base_w = wid * b_per_w

        def idx_slice(j):
            return idx_hbm.at[pl.ds(base_w + j * chunk, chunk)]

        def out_slice(j):
            return out_hbm.at[pl.ds(base_w + j * chunk, chunk), :]

        def conn_start(j):
            pltpu.async_copy(idx_slice(j), idx_v[j % NBI], csem[j % NBI])

        def conn_wait(j):
            pltpu.make_async_copy(
                idx_slice(j), idx_v[j % NBI], csem[j % NBI]).wait()

        def out_start(j):
            pltpu.async_copy(rows_v[j % NBUF], out_slice(j), osem[j % NBUF])

        def out_wait(j):
            pltpu.make_async_copy(
                rows_v[j % NBUF], out_slice(j), osem[j % NBUF]).wait()

        def gather_start(j):
            pltpu.async_copy(
                table_sh.at[idx_v[j % NBI]], rows_v[j % NBUF],
                gsem[j % NBUF])

        def gather_wait(j):
            pltpu.make_async_copy(
                table_sh.at[idx_v[j % NBI]], rows_v[j % NBUF],
                gsem[j % NBUF]).wait()

        for j in range(min(NBI, n_chunks)):
            conn_start(j)

        groups = V // 8
        gbase, grem = divmod(groups, NS)
        start = 0
        for t in range(NS):
            nrows = (gbase + (1 if t < grem else 0)) * 8
            tsrc = atom_hbm.at[pl.ds(start, nrows), :]
            tdst = table_sh.at[pl.ds(start, nrows), :]

            @pl.when(sid == t)
            def _(tsrc=tsrc, tdst=tdst):
                pltpu.async_copy(tsrc, tdst, ts)
                pltpu.make_async_copy(tsrc, tdst, ts).wait()

            start += nrows
        plsc.subcore_barrier()

        for j in range(n_chunks):
            conn_wait(j)
            if j >= NBUF:
                out_wait(j - NBUF)
            gather_start(j)
            if j >= G:
                gather_wait(j - G)
                out_start(j - G)
                if j - G + NBI < n_chunks:
                    conn_start(j - G + NBI)

        for j in range(max(0, n_chunks - G), n_chunks):
            gather_wait(j)
            out_start(j)
        for j in range(max(0, n_chunks - NBUF), n_chunks):
            out_wait(j)

    return k


def kernel(atom_matrix, connectivity):
    V, D = atom_matrix.shape
    B = connectivity.shape[0]
    assert B % NW == 0
    b_per_w = B // NW
    chunk = CHUNK
    assert b_per_w % chunk == 0 and chunk % L == 0
    assert V % 8 == 0
    n_chunks = b_per_w // chunk
    idx = connectivity[:, 1].astype(jnp.int32)
    return _gather_grid(V, b_per_w, n_chunks, chunk, D)(atom_matrix, idx)
